# baseline (speedup 1.0000x reference)
import functools

import jax
import jax.numpy as jnp
from jax import lax
from jax.experimental import pallas as pl
from jax.experimental.pallas import tpu as pltpu
from jax.experimental.pallas import tpu_sc as plsc

BATCH = 16384
VOCAB = 1000000
HIDDEN = 64
NUM_SAMPLE = 5

NPAIR = BATCH * NUM_SAMPLE
NTILE = 16
PT = NPAIR // NTILE
BT = PT // NUM_SAMPLE
NJ = PT // 128
HC = HIDDEN // 2

CH = 124928
ASZ = 4 * CH
BSZ = VOCAB - ASZ
TAIL = 640


def _planes_body(inputsT_hbm, idx_hbm, tableT_hbm, tailT_hbm, part_hbm,
                 idx_v, ia_v, ib_v, hf_v, biv, val_a, val_b, acc_v,
                 inp_v, tl_v, spA, spB, sem_p, sem_g, sem_i, sem_t):
    c = lax.axis_index("c")
    s = lax.axis_index("s")
    h0 = c * HC
    b0 = s * BT

    pltpu.sync_copy(idx_hbm.at[s], idx_v)

    lane = lax.iota(jnp.int32, 16)

    def init_body(j, _):
        for l in range(8):
            sl = pl.ds(l * 16, 16)
            base = j * 128 + l * 16
            biv[j, sl] = (base + lane) // NUM_SAMPLE
            acc_v[j, sl] = jnp.zeros((16,), jnp.float32)
            r = idx_v[j, sl]
            ia_v[j, sl] = jnp.minimum(r, ASZ - 1)
            ib_v[j, sl] = jnp.maximum(r - ASZ, 0)
            hf_v[j, sl] = (r >= ASZ).astype(jnp.int32)
        return 0

    lax.fori_loop(0, NJ, init_body, 0)

    def fire_half(hh, half):
        for i in range(4):
            @pl.when(s == (i if half == 0 else 4 + i))
            def _(i=i):
                goff = (half * ASZ) + i * CH
                dst = spA if half == 0 else spB
                pltpu.async_copy(
                    tableT_hbm.at[hh].at[pl.ds(goff, CH)],
                    dst.at[pl.ds(i * CH, CH)], sem_p)
        if half == 1:
            @pl.when(s == 8)
            def _():
                pltpu.async_copy(tailT_hbm.at[hh], tl_v, sem_t)

    def drain_half(half):
        for i in range(4):
            @pl.when(s == (i if half == 0 else 4 + i))
            def _(i=i):
                goff = (half * ASZ) + i * CH
                dst = spA if half == 0 else spB
                pltpu.make_async_copy(
                    tableT_hbm.at[h0].at[pl.ds(goff, CH)],
                    dst.at[pl.ds(i * CH, CH)], sem_p).wait()
        if half == 1:
            @pl.when(s == 8)
            def _():
                pltpu.make_async_copy(tailT_hbm.at[h0], tl_v, sem_t).wait()
                pltpu.sync_copy(tl_v, spB.at[pl.ds(VOCAB - ASZ - TAIL, TAIL)])

    fire_half(h0, 0)
    fire_half(h0, 1)

    def plane_body(k, _):
        h = h0 + k

        inp_cp = pltpu.async_copy(
            inputsT_hbm.at[h, pl.ds(b0, BT)], inp_v, sem_i)

        drain_half(0)
        plsc.subcore_barrier()
        ga = [pltpu.async_copy(spA.at[ia_v.at[j]], val_a.at[j], sem_g)
              for j in range(NJ)]
        for g in ga:
            g.wait()
        plsc.subcore_barrier()

        drain_half(1)
        plsc.subcore_barrier()
        gb = [pltpu.async_copy(spB.at[ib_v.at[j]], val_b.at[j], sem_g)
              for j in range(NJ)]
        for g in gb:
            g.wait()
        plsc.subcore_barrier()

        @pl.when(k < HC - 1)
        def _():
            fire_half(h + 1, 0)
            fire_half(h + 1, 1)

        inp_cp.wait()

        def comp(j, _):
            for l in range(8):
                sl = pl.ds(l * 16, 16)
                x = plsc.load_gather(inp_v, [biv[j, sl]])
                v = jnp.where(hf_v[j, sl] == 0, val_a[j, sl], val_b[j, sl])
                acc_v[j, sl] = acc_v[j, sl] + v * x
            return 0

        lax.fori_loop(0, NJ, comp, 0)
        return 0

    lax.fori_loop(0, HC, plane_body, 0)

    pltpu.sync_copy(acc_v, part_hbm.at[c, s])


@jax.jit
def _planes(inputsT, idx3, tableT, tailT):
    mesh = plsc.VectorSubcoreMesh(core_axis_name="c", subcore_axis_name="s")
    f = pl.kernel(
        _planes_body,
        mesh=mesh,
        out_type=jax.ShapeDtypeStruct((2, NTILE, NJ, 128), jnp.float32),
        scratch_types=[
            pltpu.VMEM((NJ, 128), jnp.int32),
            pltpu.VMEM((NJ, 128), jnp.int32),
            pltpu.VMEM((NJ, 128), jnp.int32),
            pltpu.VMEM((NJ, 128), jnp.int32),
            pltpu.VMEM((NJ, 128), jnp.int32),
            pltpu.VMEM((NJ, 128), jnp.float32),
            pltpu.VMEM((NJ, 128), jnp.float32),
            pltpu.VMEM((NJ, 128), jnp.float32),
            pltpu.VMEM((BT,), jnp.float32),
            pltpu.VMEM((TAIL,), jnp.float32),
            pltpu.VMEM_SHARED((ASZ,), jnp.float32),
            pltpu.VMEM_SHARED((BSZ,), jnp.float32),
            pltpu.SemaphoreType.DMA,
            pltpu.SemaphoreType.DMA,
            pltpu.SemaphoreType.DMA,
            pltpu.SemaphoreType.DMA,
        ],
        compiler_params=pltpu.CompilerParams(needs_layout_passes=False),
    )
    return f(inputsT, idx3, tableT, tailT)


def _combine_body(p_ref, o_ref):
    z = p_ref[0] + p_ref[1]
    o_ref[...] = 1.0 / (1.0 + jnp.exp(-z))


@jax.jit
def _combine(part):
    return pl.pallas_call(
        _combine_body,
        out_shape=jax.ShapeDtypeStruct((NPAIR // 128, 128), jnp.float32),
    )(part)


def kernel(inputs, idxs, out_embedding):
    tableT = out_embedding.T
    inputsT = inputs.T
    tailT = out_embedding[VOCAB - TAIL:, :].T
    idx3 = idxs.reshape(-1).astype(jnp.int32).reshape(NTILE, NJ, 128)
    part = _planes(inputsT, idx3, tableT, tailT)
    out = _combine(part.reshape(2, NPAIR // 128, 128))
    return out.reshape(BATCH, NUM_SAMPLE)


# speedup vs baseline: 5.6418x; 5.6418x over previous
import functools

import jax
import jax.numpy as jnp
from jax import lax
from jax.experimental import pallas as pl
from jax.experimental.pallas import tpu as pltpu
from jax.experimental.pallas import tpu_sc as plsc

BATCH = 16384
VOCAB = 1000000
HIDDEN = 64
NUM_SAMPLE = 5

NPAIR = BATCH * NUM_SAMPLE
NTILE = 16
PT = NPAIR // NTILE
BT = PT // NUM_SAMPLE
NJ = PT // 128
HC = HIDDEN // 2

CH = 124928
ASZ = 4 * CH
BSZ = VOCAB - ASZ
TAIL = 640


def _planes_body(inputsT_hbm, idx_hbm, tableT_hbm, tailT_hbm, part_hbm,
                 idx_v, ia_v, ib_v, hf_v, biv, val_a, val_b, acc_v,
                 inp_v, tl_v, spA, spB, sem_p, sem_g, sem_i, sem_t):
    c = lax.axis_index("c")
    s = lax.axis_index("s")
    h0 = c * HC
    b0 = s * BT

    pltpu.sync_copy(idx_hbm.at[s], idx_v)

    lane = lax.iota(jnp.int32, 16)

    def init_body(j, _):
        for l in range(8):
            sl = pl.ds(l * 16, 16)
            base = j * 128 + l * 16
            biv[j, sl] = (base + lane) // NUM_SAMPLE
            acc_v[j, sl] = jnp.zeros((16,), jnp.float32)
            r = idx_v[j, sl]
            hf = (r >= ASZ).astype(jnp.int32)
            t = r - hf * ASZ
            ia_v[j, sl] = jnp.minimum(t, ASZ - 1)
            ib_v[j, sl] = t
            hf_v[j, sl] = hf
        return 0

    lax.fori_loop(0, NJ, init_body, 0)

    def fire_half(hh, half):
        for i in range(4):
            @pl.when(s == (i if half == 0 else 4 + i))
            def _(i=i):
                goff = (half * ASZ) + i * CH
                dst = spA if half == 0 else spB
                pltpu.async_copy(
                    tableT_hbm.at[hh].at[pl.ds(goff, CH)],
                    dst.at[pl.ds(i * CH, CH)], sem_p)
        if half == 1:
            @pl.when(s == 8)
            def _():
                pltpu.async_copy(tailT_hbm.at[hh], tl_v, sem_t)

    def drain_half(half):
        for i in range(4):
            @pl.when(s == (i if half == 0 else 4 + i))
            def _(i=i):
                goff = (half * ASZ) + i * CH
                dst = spA if half == 0 else spB
                pltpu.make_async_copy(
                    tableT_hbm.at[h0].at[pl.ds(goff, CH)],
                    dst.at[pl.ds(i * CH, CH)], sem_p).wait()
        if half == 1:
            @pl.when(s == 8)
            def _():
                pltpu.make_async_copy(tailT_hbm.at[h0], tl_v, sem_t).wait()
                pltpu.sync_copy(tl_v, spB.at[pl.ds(VOCAB - ASZ - TAIL, TAIL)])

    fire_half(h0, 0)
    fire_half(h0, 1)

    def plane_body(k, _):
        h = h0 + k

        inp_cp = pltpu.async_copy(
            inputsT_hbm.at[h, pl.ds(b0, BT)], inp_v, sem_i)

        drain_half(0)
        plsc.subcore_barrier()
        ga = [pltpu.async_copy(spA.at[ia_v.at[j]], val_a.at[j], sem_g)
              for j in range(NJ)]
        for g in ga:
            g.wait()
        plsc.subcore_barrier()

        drain_half(1)
        plsc.subcore_barrier()
        gb = [pltpu.async_copy(spB.at[ib_v.at[j]], val_b.at[j], sem_g)
              for j in range(NJ)]
        for g in gb:
            g.wait()
        plsc.subcore_barrier()

        @pl.when(k < HC - 1)
        def _():
            fire_half(h + 1, 0)
            fire_half(h + 1, 1)

        inp_cp.wait()

        def comp(j, _):
            for l in range(8):
                sl = pl.ds(l * 16, 16)
                x = plsc.load_gather(inp_v, [biv[j, sl]])
                v = jnp.where(hf_v[j, sl] == 0, val_a[j, sl], val_b[j, sl])
                acc_v[j, sl] = acc_v[j, sl] + v * x
            return 0

        lax.fori_loop(0, NJ, comp, 0)
        return 0

    lax.fori_loop(0, HC, plane_body, 0)

    pltpu.sync_copy(acc_v, part_hbm.at[c, s])


@jax.jit
def _planes(inputsT, idx3, tableT, tailT):
    mesh = plsc.VectorSubcoreMesh(core_axis_name="c", subcore_axis_name="s")
    f = pl.kernel(
        _planes_body,
        mesh=mesh,
        out_type=jax.ShapeDtypeStruct((2, NTILE, NJ, 128), jnp.float32),
        scratch_types=[
            pltpu.VMEM((NJ, 128), jnp.int32),
            pltpu.VMEM((NJ, 128), jnp.int32),
            pltpu.VMEM((NJ, 128), jnp.int32),
            pltpu.VMEM((NJ, 128), jnp.int32),
            pltpu.VMEM((NJ, 128), jnp.int32),
            pltpu.VMEM((NJ, 128), jnp.float32),
            pltpu.VMEM((NJ, 128), jnp.float32),
            pltpu.VMEM((NJ, 128), jnp.float32),
            pltpu.VMEM((BT,), jnp.float32),
            pltpu.VMEM((TAIL,), jnp.float32),
            pltpu.VMEM_SHARED((ASZ,), jnp.float32),
            pltpu.VMEM_SHARED((BSZ,), jnp.float32),
            pltpu.SemaphoreType.DMA,
            pltpu.SemaphoreType.DMA,
            pltpu.SemaphoreType.DMA,
            pltpu.SemaphoreType.DMA,
        ],
        compiler_params=pltpu.CompilerParams(needs_layout_passes=False),
    )
    return f(inputsT, idx3, tableT, tailT)


def _combine_body(p_ref, o_ref):
    z = p_ref[0] + p_ref[1]
    o_ref[...] = 1.0 / (1.0 + jnp.exp(-z))


@jax.jit
def _combine(part):
    return pl.pallas_call(
        _combine_body,
        out_shape=jax.ShapeDtypeStruct((NPAIR // 128, 128), jnp.float32),
    )(part)


def kernel(inputs, idxs, out_embedding):
    tableT = out_embedding.T
    inputsT = inputs.T
    tailT = out_embedding[VOCAB - TAIL:, :].T
    idx3 = idxs.reshape(-1).astype(jnp.int32).reshape(NTILE, NJ, 128)
    part = _planes(inputsT, idx3, tableT, tailT)
    out = _combine(part.reshape(2, NPAIR // 128, 128))
    return out.reshape(BATCH, NUM_SAMPLE)


# speedup vs baseline: 7.1105x; 1.2603x over previous
import functools

import jax
import jax.numpy as jnp
from jax import lax
from jax.experimental import pallas as pl
from jax.experimental.pallas import tpu as pltpu
from jax.experimental.pallas import tpu_sc as plsc

BATCH = 16384
VOCAB = 1000000
HIDDEN = 64
NUM_SAMPLE = 5

NPAIR = BATCH * NUM_SAMPLE
NTILE = 16
PT = NPAIR // NTILE
BT = PT // NUM_SAMPLE
NJ = PT // 128
HC = HIDDEN // 2

CH = 124928
ASZ = 4 * CH
BSZ = VOCAB - ASZ
TAIL = 640


def _planes_body(inputsT_hbm, idx_hbm, tableT_hbm, tailT_hbm, part_hbm,
                 idx_v, ia_v, ib_v, hf_v, biv, val_a, val_b, acc_v,
                 inp_v, tl_v, spA, spB, sem_p, sem_g, sem_i, sem_t):
    c = lax.axis_index("c")
    s = lax.axis_index("s")
    h0 = c * HC
    b0 = s * BT

    pltpu.sync_copy(idx_hbm.at[s], idx_v)

    lane = lax.iota(jnp.int32, 16)

    def init_body(j, _):
        for l in range(8):
            sl = pl.ds(l * 16, 16)
            base = j * 128 + l * 16
            biv[j, sl] = (base + lane) // NUM_SAMPLE
            acc_v[j, sl] = jnp.zeros((16,), jnp.float32)
            r = idx_v[j, sl]
            hf = (r >= ASZ).astype(jnp.int32)
            t = r - hf * ASZ
            ia_v[j, sl] = jnp.minimum(t, ASZ - 1)
            ib_v[j, sl] = t
            hf_v[j, sl] = hf
        return 0

    lax.fori_loop(0, NJ, init_body, 0)

    def fire_half(hh, half):
        for i in range(4):
            @pl.when(s == (i if half == 0 else 4 + i))
            def _(i=i):
                goff = (half * ASZ) + i * CH
                dst = spA if half == 0 else spB
                pltpu.async_copy(
                    tableT_hbm.at[hh].at[pl.ds(goff, CH)],
                    dst.at[pl.ds(i * CH, CH)], sem_p)
        if half == 1:
            @pl.when(s == 8)
            def _():
                pltpu.async_copy(tailT_hbm.at[hh], tl_v, sem_t)

    def drain_half(half):
        for i in range(4):
            @pl.when(s == (i if half == 0 else 4 + i))
            def _(i=i):
                goff = (half * ASZ) + i * CH
                dst = spA if half == 0 else spB
                pltpu.make_async_copy(
                    tableT_hbm.at[h0].at[pl.ds(goff, CH)],
                    dst.at[pl.ds(i * CH, CH)], sem_p).wait()
        if half == 1:
            @pl.when(s == 8)
            def _():
                pltpu.make_async_copy(tailT_hbm.at[h0], tl_v, sem_t).wait()
                pltpu.sync_copy(tl_v, spB.at[pl.ds(VOCAB - ASZ - TAIL, TAIL)])

    fire_half(h0, 0)
    fire_half(h0, 1)

    def plane_body(k, _):
        h = h0 + k

        inp_cp = pltpu.async_copy(
            inputsT_hbm.at[h, pl.ds(b0, BT)], inp_v, sem_i)

        drain_half(0)
        plsc.subcore_barrier()
        ga = [pltpu.async_copy(spA.at[ia_v.at[j]], val_a.at[j], sem_g)
              for j in range(NJ)]
        for g in ga:
            g.wait()
        plsc.subcore_barrier()

        @pl.when(k < HC - 1)
        def _():
            fire_half(h + 1, 0)

        drain_half(1)
        plsc.subcore_barrier()
        gb = [pltpu.async_copy(spB.at[ib_v.at[j]], val_b.at[j], sem_g)
              for j in range(NJ)]
        for g in gb:
            g.wait()
        plsc.subcore_barrier()

        @pl.when(k < HC - 1)
        def _():
            fire_half(h + 1, 1)

        inp_cp.wait()

        def comp(j, _):
            for l in range(8):
                sl = pl.ds(l * 16, 16)
                x = plsc.load_gather(inp_v, [biv[j, sl]])
                v = jnp.where(hf_v[j, sl] == 0, val_a[j, sl], val_b[j, sl])
                acc_v[j, sl] = acc_v[j, sl] + v * x
            return 0

        lax.fori_loop(0, NJ, comp, 0)
        return 0

    lax.fori_loop(0, HC, plane_body, 0)

    pltpu.sync_copy(acc_v, part_hbm.at[c, s])


@jax.jit
def _planes(inputsT, idx3, tableT, tailT):
    mesh = plsc.VectorSubcoreMesh(core_axis_name="c", subcore_axis_name="s")
    f = pl.kernel(
        _planes_body,
        mesh=mesh,
        out_type=jax.ShapeDtypeStruct((2, NTILE, NJ, 128), jnp.float32),
        scratch_types=[
            pltpu.VMEM((NJ, 128), jnp.int32),
            pltpu.VMEM((NJ, 128), jnp.int32),
            pltpu.VMEM((NJ, 128), jnp.int32),
            pltpu.VMEM((NJ, 128), jnp.int32),
            pltpu.VMEM((NJ, 128), jnp.int32),
            pltpu.VMEM((NJ, 128), jnp.float32),
            pltpu.VMEM((NJ, 128), jnp.float32),
            pltpu.VMEM((NJ, 128), jnp.float32),
            pltpu.VMEM((BT,), jnp.float32),
            pltpu.VMEM((TAIL,), jnp.float32),
            pltpu.VMEM_SHARED((ASZ,), jnp.float32),
            pltpu.VMEM_SHARED((BSZ,), jnp.float32),
            pltpu.SemaphoreType.DMA,
            pltpu.SemaphoreType.DMA,
            pltpu.SemaphoreType.DMA,
            pltpu.SemaphoreType.DMA,
        ],
        compiler_params=pltpu.CompilerParams(needs_layout_passes=False),
    )
    return f(inputsT, idx3, tableT, tailT)


def _combine_body(p_ref, o_ref):
    z = p_ref[0] + p_ref[1]
    o_ref[...] = 1.0 / (1.0 + jnp.exp(-z))


@jax.jit
def _combine(part):
    return pl.pallas_call(
        _combine_body,
        out_shape=jax.ShapeDtypeStruct((NPAIR // 128, 128), jnp.float32),
    )(part)


def kernel(inputs, idxs, out_embedding):
    tableT = out_embedding.T
    inputsT = inputs.T
    tailT = out_embedding[VOCAB - TAIL:, :].T
    idx3 = idxs.reshape(-1).astype(jnp.int32).reshape(NTILE, NJ, 128)
    part = _planes(inputsT, idx3, tableT, tailT)
    out = _combine(part.reshape(2, NPAIR // 128, 128))
    return out.reshape(BATCH, NUM_SAMPLE)


# speedup vs baseline: 7.1159x; 1.0008x over previous
import jax
import jax.numpy as jnp
from jax import lax
from jax.experimental import pallas as pl
from jax.experimental.pallas import tpu as pltpu
from jax.experimental.pallas import tpu_sc as plsc

BATCH = 16384
VOCAB = 1000000
HIDDEN = 64
NUM_SAMPLE = 5

NPAIR = BATCH * NUM_SAMPLE
NTILE = 16
PT = NPAIR // NTILE
BT = PT // NUM_SAMPLE
NJ = PT // 128
HC = HIDDEN // 2

CH = 124928
ASZ = 4 * CH
BSZ = VOCAB - ASZ
TAIL = 640


def _planes_body(inputsT_hbm, idx_hbm, tableT_hbm, tailT_hbm, part_hbm,
                 idx_v, ia_v, ib_v, hf_v, biv, val_a, val_b, acc_v,
                 inp_v, tl_v, spA, spB, sem_p, sem_g, sem_i, sem_t):
    c = lax.axis_index("c")
    s = lax.axis_index("s")
    h0 = c * HC
    b0 = s * BT

    pltpu.sync_copy(idx_hbm.at[s], idx_v)

    lane = lax.iota(jnp.int32, 16)

    def init_body(j, _):
        for l in range(8):
            sl = pl.ds(l * 16, 16)
            base = j * 128 + l * 16
            biv[j, sl] = (base + lane) // NUM_SAMPLE
            acc_v[j, sl] = jnp.zeros((16,), jnp.float32)
            r = idx_v[j, sl]
            hf = (r >= ASZ).astype(jnp.int32)
            t = r - hf * ASZ
            ia_v[j, sl] = jnp.minimum(t, ASZ - 1)
            ib_v[j, sl] = t
            hf_v[j, sl] = hf
        return 0

    lax.fori_loop(0, NJ, init_body, 0)

    def fire_half(hh, half):
        for i in range(4):
            @pl.when(s == (i if half == 0 else 4 + i))
            def _(i=i):
                goff = (half * ASZ) + i * CH
                dst = spA if half == 0 else spB
                pltpu.async_copy(
                    tableT_hbm.at[hh].at[pl.ds(goff, CH)],
                    dst.at[pl.ds(i * CH, CH)], sem_p)
        if half == 1:
            @pl.when(s == 8)
            def _():
                pltpu.async_copy(tailT_hbm.at[hh], tl_v, sem_t)

    def drain_half(half):
        for i in range(4):
            @pl.when(s == (i if half == 0 else 4 + i))
            def _(i=i):
                goff = (half * ASZ) + i * CH
                dst = spA if half == 0 else spB
                pltpu.make_async_copy(
                    tableT_hbm.at[h0].at[pl.ds(goff, CH)],
                    dst.at[pl.ds(i * CH, CH)], sem_p).wait()
        if half == 1:
            @pl.when(s == 8)
            def _():
                pltpu.make_async_copy(tailT_hbm.at[h0], tl_v, sem_t).wait()
                pltpu.sync_copy(tl_v, spB.at[pl.ds(VOCAB - ASZ - TAIL, TAIL)])

    fire_half(h0, 0)
    fire_half(h0, 1)

    def plane_body(k, _):
        h = h0 + k

        inp_cp = pltpu.async_copy(
            inputsT_hbm.at[h, pl.ds(b0, BT)], inp_v, sem_i)

        drain_half(0)
        plsc.subcore_barrier()
        ga = [pltpu.async_copy(spA.at[ia_v.at[j]], val_a.at[j], sem_g)
              for j in range(NJ)]
        for g in ga:
            g.wait()
        plsc.subcore_barrier()

        @pl.when(k < HC - 1)
        def _():
            fire_half(h + 1, 0)

        drain_half(1)
        plsc.subcore_barrier()
        gb = [pltpu.async_copy(spB.at[ib_v.at[j]], val_b.at[j], sem_g)
              for j in range(NJ)]
        for g in gb:
            g.wait()
        plsc.subcore_barrier()

        @pl.when(k < HC - 1)
        def _():
            fire_half(h + 1, 1)

        inp_cp.wait()

        def comp(j, _):
            for l in range(8):
                sl = pl.ds(l * 16, 16)
                x = plsc.load_gather(inp_v, [biv[j, sl]])
                v = jnp.where(hf_v[j, sl] == 0, val_a[j, sl], val_b[j, sl])
                acc_v[j, sl] = acc_v[j, sl] + v * x
            return 0

        lax.fori_loop(0, NJ, comp, 0)
        return 0

    lax.fori_loop(0, HC, plane_body, 0)

    pltpu.sync_copy(acc_v, part_hbm.at[c, s])


@jax.jit
def _planes(inputsT, idx3, tableT, tailT):
    mesh = plsc.VectorSubcoreMesh(core_axis_name="c", subcore_axis_name="s")
    f = pl.kernel(
        _planes_body,
        mesh=mesh,
        out_type=jax.ShapeDtypeStruct((2, NTILE, NJ, 128), jnp.float32),
        scratch_types=[
            pltpu.VMEM((NJ, 128), jnp.int32),
            pltpu.VMEM((NJ, 128), jnp.int32),
            pltpu.VMEM((NJ, 128), jnp.int32),
            pltpu.VMEM((NJ, 128), jnp.int32),
            pltpu.VMEM((NJ, 128), jnp.int32),
            pltpu.VMEM((NJ, 128), jnp.float32),
            pltpu.VMEM((NJ, 128), jnp.float32),
            pltpu.VMEM((NJ, 128), jnp.float32),
            pltpu.VMEM((BT,), jnp.float32),
            pltpu.VMEM((TAIL,), jnp.float32),
            pltpu.VMEM_SHARED((ASZ,), jnp.float32),
            pltpu.VMEM_SHARED((BSZ,), jnp.float32),
            pltpu.SemaphoreType.DMA,
            pltpu.SemaphoreType.DMA,
            pltpu.SemaphoreType.DMA,
            pltpu.SemaphoreType.DMA,
        ],
        compiler_params=pltpu.CompilerParams(needs_layout_passes=False),
    )
    return f(inputsT, idx3, tableT, tailT)


def _combine_body(p_ref, o_ref):
    z = p_ref[0] + p_ref[1]
    o_ref[...] = 1.0 / (1.0 + jnp.exp(-z))


@jax.jit
def _combine(part):
    return pl.pallas_call(
        _combine_body,
        out_shape=jax.ShapeDtypeStruct((NPAIR // 128, 128), jnp.float32),
    )(part)


def kernel(inputs, idxs, out_embedding):
    tableT = out_embedding.T
    inputsT = inputs.T
    tailT = out_embedding[VOCAB - TAIL:, :].T
    idx3 = idxs.reshape(-1).astype(jnp.int32).reshape(NTILE, NJ, 128)
    part = _planes(inputsT, idx3, tableT, tailT)
    out = _combine(part.reshape(2, NPAIR // 128, 128))
    return out.reshape(BATCH, NUM_SAMPLE)
